# speedup vs baseline: 1.8221x; 1.8221x over previous
import functools

import jax
import jax.numpy as jnp
from jax import lax
from jax.experimental import pallas as pl
from jax.experimental.pallas import tpu as pltpu
from jax.experimental.pallas import tpu_sc as plsc

B = 4
S = 2048
D = 8192
ROWS = B * S
NC = 2
NS = 16
NW = NC * NS
R_PER_W = ROWS // NW
CH = 8
NCHUNK = R_PER_W // CH

_mesh = plsc.VectorSubcoreMesh(core_axis_name="c", subcore_axis_name="s")


@functools.partial(
    pl.kernel,
    mesh=_mesh,
    out_type=jax.ShapeDtypeStruct((ROWS, D), jnp.float32),
    scratch_types=[
        pltpu.VMEM((NCHUNK, CH), jnp.int32),
        pltpu.VMEM((CH, D), jnp.float32),
        pltpu.SemaphoreType.DMA,
    ],
)
def _gather_kernel(idx_hbm, table_hbm, out_hbm, idx_v, rows_v, sem):
    wid = lax.axis_index("s") * NC + lax.axis_index("c")
    base = wid * R_PER_W
    pltpu.sync_copy(idx_hbm.at[wid], idx_v)

    def body(g, carry):
        pltpu.async_copy(table_hbm.at[idx_v.at[g]], rows_v, sem).wait()
        pltpu.sync_copy(rows_v, out_hbm.at[pl.ds(base + g * CH, CH)])
        return carry

    lax.fori_loop(0, NCHUNK, body, 0)


def kernel(x, table):
    idx = x.reshape(NW, NCHUNK, CH).astype(jnp.int32)
    out = _gather_kernel(idx, table)
    return out.reshape(B, S, D)


# speedup vs baseline: 1.9395x; 1.0644x over previous
import functools

import jax
import jax.numpy as jnp
from jax import lax
from jax.experimental import pallas as pl
from jax.experimental.pallas import tpu as pltpu
from jax.experimental.pallas import tpu_sc as plsc

B = 4
S = 2048
D = 8192
ROWS = B * S
NC = 2
NS = 16
NW = NC * NS
R_PER_W = ROWS // NW
CH = 4
NCHUNK = R_PER_W // CH
NPAIR = NCHUNK // 2

_mesh = plsc.VectorSubcoreMesh(core_axis_name="c", subcore_axis_name="s")


@functools.partial(
    pl.kernel,
    mesh=_mesh,
    out_type=jax.ShapeDtypeStruct((ROWS, D), jnp.float32),
    scratch_types=[
        pltpu.VMEM((NCHUNK, CH), jnp.int32),
        pltpu.VMEM((CH, D), jnp.float32),
        pltpu.VMEM((CH, D), jnp.float32),
        pltpu.SemaphoreType.DMA,
        pltpu.SemaphoreType.DMA,
        pltpu.SemaphoreType.DMA,
        pltpu.SemaphoreType.DMA,
    ],
)
def _gather_kernel(idx_hbm, table_hbm, out_hbm, idx_v, buf0, buf1,
                   semg0, semg1, semo0, semo1):
    wid = lax.axis_index("s") * NC + lax.axis_index("c")
    base = wid * R_PER_W
    pltpu.sync_copy(idx_hbm.at[wid], idx_v)

    pltpu.async_copy(table_hbm.at[idx_v.at[0]], buf0, semg0)

    def body(i, carry):
        g0 = 2 * i
        pltpu.make_async_copy(table_hbm.at[idx_v.at[g0]], buf0, semg0).wait()
        pltpu.async_copy(buf0, out_hbm.at[pl.ds(base + g0 * CH, CH)], semo0)
        @pl.when(i > 0)
        def _():
            pltpu.make_async_copy(
                buf1, out_hbm.at[pl.ds(base, CH)], semo1).wait()
        pltpu.async_copy(table_hbm.at[idx_v.at[g0 + 1]], buf1, semg1)
        pltpu.make_async_copy(table_hbm.at[idx_v.at[g0 + 1]], buf1, semg1).wait()
        pltpu.async_copy(
            buf1, out_hbm.at[pl.ds(base + (g0 + 1) * CH, CH)], semo1)
        pltpu.make_async_copy(buf0, out_hbm.at[pl.ds(base, CH)], semo0).wait()
        @pl.when(i < NPAIR - 1)
        def _():
            pltpu.async_copy(table_hbm.at[idx_v.at[g0 + 2]], buf0, semg0)
        return carry

    lax.fori_loop(0, NPAIR, body, 0)
    pltpu.make_async_copy(buf1, out_hbm.at[pl.ds(base, CH)], semo1).wait()


def kernel(x, table):
    idx = x.reshape(NW, NCHUNK, CH).astype(jnp.int32)
    out = _gather_kernel(idx, table)
    return out.reshape(B, S, D)
